# baseline (speedup 1.0000x reference)
import jax.numpy as jnp
F = 26; V = 100000; K = 16; B = 4096

def kernel(inputs, emb_table, lin_table, lin_bias, W1, b1, W2, b2, W3, b3):
    r = emb_table.reshape(F * V, K)
    return (r[:B, :1] + r[B:2*B, :1] * 0.0)


# speedup vs baseline: 5.1795x; 5.1795x over previous
import functools

import jax
import jax.numpy as jnp
from jax import lax
from jax.experimental import pallas as pl
from jax.experimental.pallas import tpu as pltpu
from jax.experimental.pallas import tpu_sc as plsc

F = 26
V = 100000
K = 16
B = 4096
H1, H2 = 400, 400
D0 = F * K

NC, NS = 2, 16
NW = NC * NS
PW = (B * F) // NW
CH = 128
CPW = PW // CH

C = 32
NCHUNK = PW // C
NBUF = 3
NSUPER = NCHUNK + NBUF
NOUTER = -(-NSUPER // NBUF)


def _sc_gather_body(blk_hbm, l_hbm, fidx_hbm, emb_hbm, lin_hbm,
                    emb_out, lin_out,
                    blk_v, l_v, fidx_v, lin_v, bufs, outs,
                    sem_lin, sem_g0, sem_g1, sem_g2, sem_w0, sem_w1, sem_w2):
    wid = lax.axis_index("s") * NC + lax.axis_index("c")
    base = pl.multiple_of(wid * PW, CH)
    emb3 = emb_hbm.reshape((F * V) // 8, 8, K)
    sem_g = (sem_g0, sem_g1, sem_g2)
    sem_w = (sem_w0, sem_w1, sem_w2)

    pltpu.sync_copy(blk_hbm.at[pl.ds(base, PW)], blk_v)
    pltpu.sync_copy(l_hbm.at[pl.ds(base, PW)], l_v)
    pltpu.sync_copy(fidx_hbm.at[pl.ds(base, PW)], fidx_v)

    def lin_fire(j, carry):
        off = pl.multiple_of(j * CH, CH)
        pltpu.async_copy(lin_hbm.at[fidx_v.at[pl.ds(off, CH)]],
                         lin_v.at[pl.ds(off, CH)], sem_lin)
        return carry

    lax.fori_loop(0, CPW, lin_fire, 0)

    def superstep(r, b):
        @pl.when(jnp.logical_and(r >= NBUF, r < NCHUNK + NBUF))
        def _():
            rs = r - NBUF
            offs = pl.multiple_of(rs * C, C)
            for i in range(C):
                pltpu.make_async_copy(emb3.at[0], bufs.at[b, i],
                                      sem_g[b]).wait()
            @pl.when(rs >= NBUF)
            def _():
                pltpu.make_async_copy(outs.at[b],
                                      emb_out.at[pl.ds(0, C)], sem_w[b]).wait()
            for g in range(C // 16):
                lvec = l_v[pl.ds(offs + g * 16, 16)]
                for i in range(16):
                    jj = g * 16 + i
                    outs[b, jj] = bufs[b, jj, lvec[i]]
            pltpu.async_copy(outs.at[b],
                             emb_out.at[pl.ds(base + offs, C)], sem_w[b])

        @pl.when(r < NCHUNK)
        def _():
            off = pl.multiple_of(r * C, C)
            for g in range(C // 16):
                bvec = blk_v[pl.ds(off + g * 16, 16)]
                for i in range(16):
                    pltpu.async_copy(emb3.at[bvec[i]],
                                     bufs.at[b, g * 16 + i], sem_g[b])

    def outer(ro, carry):
        for b in range(NBUF):
            superstep(ro * NBUF + b, b)
        return carry

    lax.fori_loop(0, NOUTER, outer, 0)

    for b in range(NBUF):
        pltpu.make_async_copy(outs.at[b], emb_out.at[pl.ds(0, C)],
                              sem_w[b]).wait()

    def lin_drain(j, carry):
        off = pl.multiple_of(j * CH, CH)
        pltpu.make_async_copy(lin_hbm.at[fidx_v.at[pl.ds(off, CH)]],
                              lin_v.at[pl.ds(off, CH)], sem_lin).wait()
        return carry

    lax.fori_loop(0, CPW, lin_drain, 0)
    pltpu.sync_copy(lin_v, lin_out.at[pl.ds(base, PW)])


_sc_gather = functools.partial(
    pl.kernel,
    mesh=plsc.VectorSubcoreMesh(core_axis_name="c", subcore_axis_name="s",
                                num_cores=NC, num_subcores=NS),
    out_type=[
        jax.ShapeDtypeStruct((B * F, K), jnp.float32),
        jax.ShapeDtypeStruct((B * F,), jnp.float32),
    ],
    scratch_types=[
        pltpu.VMEM((PW,), jnp.int32),
        pltpu.VMEM((PW,), jnp.int32),
        pltpu.VMEM((PW,), jnp.int32),
        pltpu.VMEM((PW,), jnp.float32),
        pltpu.VMEM((NBUF, C, 8, K), jnp.float32),
        pltpu.VMEM((NBUF, C, K), jnp.float32),
        pltpu.SemaphoreType.DMA,
        pltpu.SemaphoreType.DMA,
        pltpu.SemaphoreType.DMA,
        pltpu.SemaphoreType.DMA,
        pltpu.SemaphoreType.DMA,
        pltpu.SemaphoreType.DMA,
        pltpu.SemaphoreType.DMA,
    ],
)(_sc_gather_body)


BB = 512


def _mlp_body(x_ref, lin_ref, m_ref, w1_ref, b1_ref, w2_ref, b2_ref,
              w3_ref, b3_ref, lb_ref, out_ref, inter_ref, acc_ref):
    i = pl.program_id(0)
    x = x_ref[...]
    s = jnp.dot(x, m_ref[...], precision=lax.Precision.HIGHEST)
    part = 0.5 * (jnp.sum(s * s, axis=(0, 1), keepdims=True)
                  - jnp.sum(x * x, axis=(0, 1), keepdims=True))

    @pl.when(i == 0)
    def _():
        acc_ref[...] = jnp.zeros((1, 1), jnp.float32)

    acc_ref[...] += part
    h = jnp.maximum(
        jnp.dot(x, w1_ref[...], precision=lax.Precision.HIGHEST) + b1_ref[...], 0.0)
    h = jnp.maximum(
        jnp.dot(h, w2_ref[...], precision=lax.Precision.HIGHEST) + b2_ref[...], 0.0)
    fnn = jnp.dot(h, w3_ref[...], precision=lax.Precision.HIGHEST) + b3_ref[...]
    line = jnp.sum(lin_ref[...], axis=1, keepdims=True) + lb_ref[...]
    out_ref[...] = line + fnn
    inter_ref[...] = acc_ref[...]


_mlp = pl.pallas_call(
    _mlp_body,
    grid=(B // BB,),
    in_specs=[
        pl.BlockSpec((BB, D0), lambda i: (i, 0)),
        pl.BlockSpec((BB, F), lambda i: (i, 0)),
        pl.BlockSpec((D0, K), lambda i: (0, 0)),
        pl.BlockSpec((D0, H1), lambda i: (0, 0)),
        pl.BlockSpec((1, H1), lambda i: (0, 0)),
        pl.BlockSpec((H1, H2), lambda i: (0, 0)),
        pl.BlockSpec((1, H2), lambda i: (0, 0)),
        pl.BlockSpec((H2, 1), lambda i: (0, 0)),
        pl.BlockSpec((1, 1), lambda i: (0, 0)),
        pl.BlockSpec((1, 1), lambda i: (0, 0)),
    ],
    out_specs=[
        pl.BlockSpec((BB, 1), lambda i: (i, 0)),
        pl.BlockSpec((1, 1), lambda i: (0, 0)),
    ],
    out_shape=[
        jax.ShapeDtypeStruct((B, 1), jnp.float32),
        jax.ShapeDtypeStruct((1, 1), jnp.float32),
    ],
    scratch_shapes=[pltpu.VMEM((1, 1), jnp.float32)],
)


def kernel(inputs, emb_table, lin_table, lin_bias, W1, b1, W2, b2, W3, b3):
    flat_idx = (inputs + (jnp.arange(F, dtype=jnp.int32) * V)[None, :]).reshape(B * F)
    blk_idx = flat_idx // 8
    l_idx = flat_idx % 8
    lin_flat = lin_table.reshape(F * V)
    emb_rows, lin_rows = _sc_gather(blk_idx, l_idx, flat_idx,
                                    emb_table, lin_flat)
    x = emb_rows.reshape(B, D0)
    lin_m = lin_rows.reshape(B, F)
    m = jnp.tile(jnp.eye(K, dtype=jnp.float32), (F, 1))
    base, inter = _mlp(x, lin_m, m, W1, b1.reshape(1, H1), W2,
                       b2.reshape(1, H2), W3, b3.reshape(1, 1),
                       lin_bias.reshape(1, 1))
    return base + inter


# speedup vs baseline: 19.7331x; 3.8098x over previous
import functools

import jax
import jax.numpy as jnp
from jax import lax
from jax.experimental import pallas as pl
from jax.experimental.pallas import tpu as pltpu
from jax.experimental.pallas import tpu_sc as plsc

F = 26
V = 100000
K = 16
B = 4096
H1, H2 = 400, 400
D0 = F * K

NC, NS = 2, 16
NW = NC * NS
BPW = B // NW


PPW = D0 // NW


def _sc_gather_body(idxT_hbm, fidxT_hbm, embT_hbm, lin_hbm, xT_out, linT_out,
                    plane_v, idx_v, row_v, fidx_v, linT_v, sem_p, sem_l):
    wid = lax.axis_index("s") * NC + lax.axis_index("c")

    b0 = pl.multiple_of(wid * BPW, BPW)
    pltpu.sync_copy(fidxT_hbm.at[:, pl.ds(b0, BPW)], fidx_v)

    def lin_fire(f, carry):
        pltpu.async_copy(lin_hbm.at[fidx_v.at[f]], linT_v.at[f], sem_l)
        return carry

    lax.fori_loop(0, F, lin_fire, 0)

    def plane(i, carry):
        p = wid * PPW + i
        f = p // K
        k = p % K
        pltpu.sync_copy(embT_hbm.at[f, k], plane_v)
        pltpu.sync_copy(idxT_hbm.at[f], idx_v)

        def grp(g, c2):
            off = pl.multiple_of(g * 16, 16)
            vvec = idx_v[pl.ds(off, 16)]
            row_v[pl.ds(off, 16)] = plsc.load_gather(plane_v, [vvec])
            return c2

        lax.fori_loop(0, B // 16, grp, 0)
        pltpu.sync_copy(row_v, xT_out.at[p])
        return carry

    lax.fori_loop(0, PPW, plane, 0)

    def lin_drain(f, carry):
        pltpu.make_async_copy(lin_hbm.at[fidx_v.at[f]],
                              linT_v.at[f], sem_l).wait()
        return carry

    lax.fori_loop(0, F, lin_drain, 0)
    pltpu.sync_copy(linT_v, linT_out.at[:, pl.ds(b0, BPW)])


_sc_gather = functools.partial(
    pl.kernel,
    mesh=plsc.VectorSubcoreMesh(core_axis_name="c", subcore_axis_name="s",
                                num_cores=NC, num_subcores=NS),
    compiler_params=pltpu.CompilerParams(needs_layout_passes=False),
    out_type=[
        jax.ShapeDtypeStruct((D0, B), jnp.float32),
        jax.ShapeDtypeStruct((F, B), jnp.float32),
    ],
    scratch_types=[
        pltpu.VMEM((V,), jnp.float32),
        pltpu.VMEM((B,), jnp.int32),
        pltpu.VMEM((B,), jnp.float32),
        pltpu.VMEM((F, BPW), jnp.int32),
        pltpu.VMEM((F, BPW), jnp.float32),
        pltpu.SemaphoreType.DMA,
        pltpu.SemaphoreType.DMA,
    ],
)(_sc_gather_body)


BB = 512


def _mlp_body(xt_ref, lin_ref, m_ref, w1t_ref, b1_ref, w2t_ref, b2_ref,
              w3t_ref, b3_ref, lb_ref, out_ref, inter_ref, acc_ref):
    i = pl.program_id(0)
    xt = xt_ref[...]
    st = jnp.dot(m_ref[...], xt, precision=lax.Precision.HIGHEST)
    part = 0.5 * (jnp.sum(st * st, axis=(0, 1), keepdims=True)
                  - jnp.sum(xt * xt, axis=(0, 1), keepdims=True))

    @pl.when(i == 0)
    def _():
        acc_ref[...] = jnp.zeros((1, 1), jnp.float32)

    acc_ref[...] += part
    h = jnp.maximum(
        jnp.dot(w1t_ref[...], xt, precision=lax.Precision.HIGHEST)
        + b1_ref[...], 0.0)
    h = jnp.maximum(
        jnp.dot(w2t_ref[...], h, precision=lax.Precision.HIGHEST)
        + b2_ref[...], 0.0)
    fnn = jnp.dot(w3t_ref[...], h, precision=lax.Precision.HIGHEST) + b3_ref[...]
    line = jnp.sum(lin_ref[...], axis=0, keepdims=True) + lb_ref[...]
    out_ref[...] = line + fnn
    inter_ref[...] = acc_ref[...]


_mlp = pl.pallas_call(
    _mlp_body,
    grid=(B // BB,),
    in_specs=[
        pl.BlockSpec((D0, BB), lambda i: (0, i)),
        pl.BlockSpec((F, BB), lambda i: (0, i)),
        pl.BlockSpec((K, D0), lambda i: (0, 0)),
        pl.BlockSpec((H1, D0), lambda i: (0, 0)),
        pl.BlockSpec((H1, 1), lambda i: (0, 0)),
        pl.BlockSpec((H2, H1), lambda i: (0, 0)),
        pl.BlockSpec((H2, 1), lambda i: (0, 0)),
        pl.BlockSpec((1, H2), lambda i: (0, 0)),
        pl.BlockSpec((1, 1), lambda i: (0, 0)),
        pl.BlockSpec((1, 1), lambda i: (0, 0)),
    ],
    out_specs=[
        pl.BlockSpec((1, BB), lambda i: (0, i)),
        pl.BlockSpec((1, 1), lambda i: (0, 0)),
    ],
    out_shape=[
        jax.ShapeDtypeStruct((1, B), jnp.float32),
        jax.ShapeDtypeStruct((1, 1), jnp.float32),
    ],
    scratch_shapes=[pltpu.VMEM((1, 1), jnp.float32)],
)


def kernel(inputs, emb_table, lin_table, lin_bias, W1, b1, W2, b2, W3, b3):
    idxT = inputs.T
    fidxT = idxT + (jnp.arange(F, dtype=jnp.int32) * V)[:, None]
    embT = jnp.transpose(emb_table, (0, 2, 1))
    lin_flat = lin_table.reshape(F * V)
    xT, linT = _sc_gather(idxT, fidxT, embT, lin_flat)
    msel = jnp.tile(jnp.eye(K, dtype=jnp.float32), (1, F))
    outT, inter = _mlp(xT, linT, msel, W1.T, b1.reshape(H1, 1), W2.T,
                       b2.reshape(H2, 1), W3.T, b3.reshape(1, 1),
                       lin_bias.reshape(1, 1))
    return outT.reshape(B, 1) + inter


# speedup vs baseline: 21.3078x; 1.0798x over previous
import functools

import jax
import jax.numpy as jnp
from jax import lax
from jax.experimental import pallas as pl
from jax.experimental.pallas import tpu as pltpu
from jax.experimental.pallas import tpu_sc as plsc

F = 26
V = 100000
K = 16
B = 4096
H1, H2 = 400, 400
D0 = F * K

NC, NS = 2, 16
NW = NC * NS
BPW = B // NW


PPW = D0 // NW


def _sc_gather_body(idxT_hbm, fidxT_hbm, embT_hbm, lin_hbm, xT_out, linT_out,
                    plane_v, idx_v, row_v, fidx_v, linT_v, sem_p, sem_l):
    wid = lax.axis_index("s") * NC + lax.axis_index("c")

    b0 = pl.multiple_of(wid * BPW, BPW)
    pltpu.sync_copy(fidxT_hbm.at[:, pl.ds(b0, BPW)], fidx_v)

    def lin_fire(f, carry):
        pltpu.async_copy(lin_hbm.at[fidx_v.at[f]], linT_v.at[f], sem_l)
        return carry

    lax.fori_loop(0, F, lin_fire, 0)

    def plane(i, carry):
        p = wid * PPW + i
        f = p // K
        k = p % K
        pltpu.sync_copy(embT_hbm.at[f, k], plane_v)
        pltpu.sync_copy(idxT_hbm.at[f], idx_v)

        def grp(g, c2):
            off = pl.multiple_of(g * 16, 16)
            vvec = idx_v[pl.ds(off, 16)]
            row_v[pl.ds(off, 16)] = plsc.load_gather(plane_v, [vvec])
            return c2

        lax.fori_loop(0, B // 16, grp, 0)
        pltpu.sync_copy(row_v, xT_out.at[p])
        return carry

    lax.fori_loop(0, PPW, plane, 0)

    def lin_drain(f, carry):
        pltpu.make_async_copy(lin_hbm.at[fidx_v.at[f]],
                              linT_v.at[f], sem_l).wait()
        return carry

    lax.fori_loop(0, F, lin_drain, 0)
    pltpu.sync_copy(linT_v, linT_out.at[:, pl.ds(b0, BPW)])


_sc_gather = functools.partial(
    pl.kernel,
    mesh=plsc.VectorSubcoreMesh(core_axis_name="c", subcore_axis_name="s",
                                num_cores=NC, num_subcores=NS),
    compiler_params=pltpu.CompilerParams(needs_layout_passes=False),
    out_type=[
        jax.ShapeDtypeStruct((D0, B), jnp.float32),
        jax.ShapeDtypeStruct((F, B), jnp.float32),
    ],
    scratch_types=[
        pltpu.VMEM((V,), jnp.float32),
        pltpu.VMEM((B,), jnp.int32),
        pltpu.VMEM((B,), jnp.float32),
        pltpu.VMEM((F, BPW), jnp.int32),
        pltpu.VMEM((F, BPW), jnp.float32),
        pltpu.SemaphoreType.DMA,
        pltpu.SemaphoreType.DMA,
    ],
)(_sc_gather_body)


BB = 512


def _mlp_body(xt_ref, lin_ref, m_ref, w1t_ref, b1_ref, w2t_ref, b2_ref,
              w3t_ref, b3_ref, lb_ref, out_ref, inter_ref, acc_ref):
    i = pl.program_id(0)
    xt = xt_ref[...]
    st = jnp.dot(m_ref[...], xt, precision=lax.Precision.HIGHEST)
    part = 0.5 * (jnp.sum(st * st, axis=(0, 1), keepdims=True)
                  - jnp.sum(xt * xt, axis=(0, 1), keepdims=True))

    @pl.when(i == 0)
    def _():
        acc_ref[...] = jnp.zeros((1, 1), jnp.float32)

    acc_ref[...] += part
    h = jnp.maximum(
        jnp.dot(w1t_ref[...], xt, precision=lax.Precision.DEFAULT)
        + b1_ref[...], 0.0)
    h = jnp.maximum(
        jnp.dot(w2t_ref[...], h, precision=lax.Precision.DEFAULT)
        + b2_ref[...], 0.0)
    fnn = jnp.dot(w3t_ref[...], h, precision=lax.Precision.DEFAULT) + b3_ref[...]
    line = jnp.sum(lin_ref[...], axis=0, keepdims=True) + lb_ref[...]
    out_ref[...] = line + fnn
    inter_ref[...] = acc_ref[...]


_mlp = pl.pallas_call(
    _mlp_body,
    grid=(B // BB,),
    in_specs=[
        pl.BlockSpec((D0, BB), lambda i: (0, i)),
        pl.BlockSpec((F, BB), lambda i: (0, i)),
        pl.BlockSpec((K, D0), lambda i: (0, 0)),
        pl.BlockSpec((H1, D0), lambda i: (0, 0)),
        pl.BlockSpec((H1, 1), lambda i: (0, 0)),
        pl.BlockSpec((H2, H1), lambda i: (0, 0)),
        pl.BlockSpec((H2, 1), lambda i: (0, 0)),
        pl.BlockSpec((1, H2), lambda i: (0, 0)),
        pl.BlockSpec((1, 1), lambda i: (0, 0)),
        pl.BlockSpec((1, 1), lambda i: (0, 0)),
    ],
    out_specs=[
        pl.BlockSpec((1, BB), lambda i: (0, i)),
        pl.BlockSpec((1, 1), lambda i: (0, 0)),
    ],
    out_shape=[
        jax.ShapeDtypeStruct((1, B), jnp.float32),
        jax.ShapeDtypeStruct((1, 1), jnp.float32),
    ],
    scratch_shapes=[pltpu.VMEM((1, 1), jnp.float32)],
)


def kernel(inputs, emb_table, lin_table, lin_bias, W1, b1, W2, b2, W3, b3):
    idxT = inputs.T
    fidxT = idxT + (jnp.arange(F, dtype=jnp.int32) * V)[:, None]
    embT = jnp.transpose(emb_table, (0, 2, 1))
    lin_flat = lin_table.reshape(F * V)
    xT, linT = _sc_gather(idxT, fidxT, embT, lin_flat)
    msel = jnp.tile(jnp.eye(K, dtype=jnp.float32), (1, F))
    outT, inter = _mlp(xT, linT, msel, W1.T, b1.reshape(H1, 1), W2.T,
                       b2.reshape(H2, 1), W3.T, b3.reshape(1, 1),
                       lin_bias.reshape(1, 1))
    return outT.reshape(B, 1) + inter
